# baseline (speedup 1.0000x reference)
import functools

import jax
import jax.numpy as jnp
from jax import lax
from jax.experimental import pallas as pl
from jax.experimental.pallas import tpu as pltpu
from jax.experimental.pallas import tpu_sc as plsc

_INFO = plsc.get_sparse_core_info()
_NC, _NS = _INFO.num_cores, _INFO.num_subcores
_NW = _NC * _NS
_L = 16
_BB = 128
_PAD = 128


def _make_sc_gather(n_b, n_t, dim):
    assert n_b % (_NW * _BB) == 0 and n_t % 2 == 0 and dim == 64
    jpw = n_b // (_NW * _BB)
    assert jpw >= 2

    mesh = plsc.VectorSubcoreMesh(core_axis_name="c", subcore_axis_name="s")

    @functools.partial(
        pl.kernel,
        mesh=mesh,
        out_type=jax.ShapeDtypeStruct((n_t, dim, n_b), jnp.float32),
        scratch_types=(
            [pltpu.VMEM((n_t, _BB), jnp.int32) for _ in range(2)]
            + [pltpu.VMEM((_BB, _PAD), jnp.float32) for _ in range(2)]
            + [pltpu.VMEM((dim, _BB), jnp.float32) for _ in range(2)]
            + [pltpu.SemaphoreType.DMA] * 6
        ),
        compiler_params=pltpu.CompilerParams(needs_layout_passes=False),
    )
    def k(locT_hbm, table_hbm, out_hbm, *scr):
        idx_v = scr[0:2]
        gv = scr[2:4]
        tv = scr[4:6]
        sem_i = scr[6:8]
        sem_g = scr[8:10]
        sem_o = scr[10:12]

        wid = lax.axis_index("s") * _NC + lax.axis_index("c")
        biota = lax.iota(jnp.int32, _L)

        def b0_of(jj):
            return (wid * jpw + jj) * _BB

        def idx_load(jj, ji):
            pltpu.async_copy(
                locT_hbm.at[:, pl.ds(b0_of(jj), _BB)], idx_v[ji], sem_i[ji])

        def idx_wait(ji):
            pltpu.make_async_copy(
                locT_hbm.at[:, pl.ds(0, _BB)], idx_v[ji], sem_i[ji]).wait()

        def gather(t, ji, g):
            pltpu.async_copy(
                table_hbm.at[idx_v[ji].at[t]], gv[g], sem_g[g])

        def gather_wait(g):
            pltpu.make_async_copy(
                table_hbm.at[pl.ds(0, _BB)], gv[g], sem_g[g]).wait()

        def transpose(p):
            def body(cg, carry):
                cols = jnp.full((_L,), cg, jnp.int32)
                for g in range(_BB // _L):
                    vals = plsc.load_gather(
                        gv[p], [biota + g * _L, cols])
                    tv[p][cg, pl.ds(g * _L, _L)] = vals
                return carry
            lax.fori_loop(0, dim, body, 0, unroll=4)

        def out_write(t, jj, p):
            pltpu.async_copy(
                tv[p], out_hbm.at[t, :, pl.ds(b0_of(jj), _BB)], sem_o[p])

        def out_wait(p):
            pltpu.make_async_copy(
                tv[p], out_hbm.at[0, :, pl.ds(0, _BB)], sem_o[p]).wait()

        def step(t, jj, ji, p, drain=True, fire=None):
            gather_wait(p)
            if fire is not None:
                gather(fire[0], fire[1], 1 - p)
            if drain:
                out_wait(p)
            transpose(p)
            out_write(t, jj, p)

        idx_load(0, 0)
        idx_wait(0)
        gather(0, 0, 0)

        for jj in range(jpw):
            ji = jj % 2
            if jj + 1 < jpw:
                idx_load(jj + 1, (jj + 1) % 2)
            for t in range(2):
                step(t, jj, ji, t % 2, drain=(jj > 0),
                     fire=(t + 1, ji))
            def t_pair(t2, carry, jj=jj, ji=ji):
                for dt in range(2):
                    t = t2 * 2 + dt
                    step(t, jj, ji, dt, fire=(t + 1, ji))
                return carry
            lax.fori_loop(1, n_t // 2 - 1, t_pair, 0, unroll=False)
            step(n_t - 2, jj, ji, 0, fire=(n_t - 1, ji))
            if jj + 1 < jpw:
                idx_wait((jj + 1) % 2)
                step(n_t - 1, jj, ji, 1, fire=(0, (jj + 1) % 2))
            else:
                step(n_t - 1, jj, ji, 1, fire=None)

        out_wait(0)
        out_wait(1)

    return k


def kernel(location, embedding_matrix):
    n_b, n_t = location.shape
    v, dim = embedding_matrix.shape
    loc_t = location.astype(jnp.int32).T
    table_pad = jnp.pad(embedding_matrix, ((0, 0), (0, _PAD - dim)))
    out_t = _make_sc_gather(n_b, n_t, dim)(loc_t, table_pad)
    return jnp.transpose(out_t, (2, 0, 1))


# speedup vs baseline: 1.3528x; 1.3528x over previous
import functools

import jax
import jax.numpy as jnp
from jax import lax
from jax.experimental import pallas as pl
from jax.experimental.pallas import tpu as pltpu
from jax.experimental.pallas import tpu_sc as plsc

_INFO = plsc.get_sparse_core_info()
_NC, _NS = _INFO.num_cores, _INFO.num_subcores
_NW = _NC * _NS
_L = 16
_BB = 128
_PAD = 128


def _make_sc_gather(n_b, n_t, dim):
    assert n_b % (_NW * _BB) == 0 and n_t % 2 == 0 and dim == 64
    jpw = n_b // (_NW * _BB)
    assert jpw >= 2

    mesh = plsc.VectorSubcoreMesh(core_axis_name="c", subcore_axis_name="s")

    @functools.partial(
        pl.kernel,
        mesh=mesh,
        out_type=jax.ShapeDtypeStruct((n_t, dim, n_b), jnp.float32),
        scratch_types=(
            [pltpu.VMEM((n_t, _BB), jnp.int32) for _ in range(2)]
            + [pltpu.VMEM((_BB, _PAD), jnp.float32) for _ in range(2)]
            + [pltpu.VMEM((dim, _BB), jnp.float32) for _ in range(2)]
            + [pltpu.SemaphoreType.DMA] * 6
        ),
        compiler_params=pltpu.CompilerParams(needs_layout_passes=False),
    )
    def k(locT_hbm, table_hbm, out_hbm, *scr):
        idx_v = scr[0:2]
        gv = scr[2:4]
        tv = scr[4:6]
        sem_i = scr[6:8]
        sem_g = scr[8:10]
        sem_o = scr[10:12]

        wid = lax.axis_index("s") * _NC + lax.axis_index("c")
        biota = lax.iota(jnp.int32, _L)

        def b0_of(jj):
            return (wid * jpw + jj) * _BB

        def idx_load(jj, ji):
            pltpu.async_copy(
                locT_hbm.at[:, pl.ds(b0_of(jj), _BB)], idx_v[ji], sem_i[ji])

        def idx_wait(ji):
            pltpu.make_async_copy(
                locT_hbm.at[:, pl.ds(0, _BB)], idx_v[ji], sem_i[ji]).wait()

        def gather(t, ji, g):
            pltpu.async_copy(
                table_hbm.at[idx_v[ji].at[t]], gv[g], sem_g[g])

        def gather_wait(g):
            pltpu.make_async_copy(
                table_hbm.at[pl.ds(0, _BB)], gv[g], sem_g[g]).wait()

        def transpose(p):
            ng = _BB // _L

            def body(cg2, carry):
                vals = []
                for dc in range(2):
                    cols = jnp.full((_L,), cg2 * 2 + dc, jnp.int32)
                    for g in range(ng):
                        vals.append(plsc.load_gather(
                            gv[p], [biota + g * _L, cols]))
                for dc in range(2):
                    for g in range(ng):
                        tv[p][cg2 * 2 + dc, pl.ds(g * _L, _L)] = (
                            vals[dc * ng + g])
                return carry
            lax.fori_loop(0, dim // 2, body, 0, unroll=2)

        def out_write(t, jj, p):
            pltpu.async_copy(
                tv[p], out_hbm.at[t, :, pl.ds(b0_of(jj), _BB)], sem_o[p])

        def out_wait(p):
            pltpu.make_async_copy(
                tv[p], out_hbm.at[0, :, pl.ds(0, _BB)], sem_o[p]).wait()

        def step(t, jj, ji, p, drain=True, fire=None):
            gather_wait(p)
            if fire is not None:
                gather(fire[0], fire[1], 1 - p)
            if drain:
                out_wait(p)
            transpose(p)
            out_write(t, jj, p)

        idx_load(0, 0)
        idx_wait(0)
        gather(0, 0, 0)

        for jj in range(jpw):
            ji = jj % 2
            if jj + 1 < jpw:
                idx_load(jj + 1, (jj + 1) % 2)
            for t in range(2):
                step(t, jj, ji, t % 2, drain=(jj > 0),
                     fire=(t + 1, ji))
            def t_pair(t2, carry, jj=jj, ji=ji):
                for dt in range(2):
                    t = t2 * 2 + dt
                    step(t, jj, ji, dt, fire=(t + 1, ji))
                return carry
            lax.fori_loop(1, n_t // 2 - 1, t_pair, 0, unroll=False)
            step(n_t - 2, jj, ji, 0, fire=(n_t - 1, ji))
            if jj + 1 < jpw:
                idx_wait((jj + 1) % 2)
                step(n_t - 1, jj, ji, 1, fire=(0, (jj + 1) % 2))
            else:
                step(n_t - 1, jj, ji, 1, fire=None)

        out_wait(0)
        out_wait(1)

    return k


def kernel(location, embedding_matrix):
    n_b, n_t = location.shape
    v, dim = embedding_matrix.shape
    loc_t = location.astype(jnp.int32).T
    table_pad = jnp.pad(embedding_matrix, ((0, 0), (0, _PAD - dim)))
    out_t = _make_sc_gather(n_b, n_t, dim)(loc_t, table_pad)
    return jnp.transpose(out_t, (2, 0, 1))


# speedup vs baseline: 2.0561x; 1.5199x over previous
import functools

import jax
import jax.numpy as jnp
from jax import lax
from jax.experimental import pallas as pl
from jax.experimental.pallas import tpu as pltpu
from jax.experimental.pallas import tpu_sc as plsc

_INFO = plsc.get_sparse_core_info()
_NC, _NS = _INFO.num_cores, _INFO.num_subcores
_NW = _NC * _NS
_L = 16

_SUP = 8
_PAD = 128
_SPLITS = ((0, 128), (128, 72))


def _make_sc_gather(n_b, n_t, dim):
    assert n_b % (_NW * _SUP) == 0 and n_t == 200 and dim == 64
    b_per_w = n_b // _NW
    n_sup = b_per_w // _SUP
    assert n_sup >= 4 and n_sup % 2 == 0

    mesh = plsc.VectorSubcoreMesh(core_axis_name="c", subcore_axis_name="s")

    @functools.partial(
        pl.kernel,
        mesh=mesh,
        out_type=jax.ShapeDtypeStruct((n_b, n_t, dim), jnp.float32),
        scratch_types=(
            [pltpu.VMEM((_SUP * n_t,), jnp.int32) for _ in range(2)]
            + [pltpu.VMEM((n_t, _PAD), jnp.float32) for _ in range(2)]
            + [pltpu.VMEM((1, n_t, dim), jnp.float32) for _ in range(2)]
            + [pltpu.SemaphoreType.DMA] * 6
        ),
    )
    def k(loc_hbm, table_hbm, out_hbm, *scr):
        idx_v = scr[0:2]
        gv = scr[2:4]
        rv = scr[4:6]
        sem_i = scr[6:8]
        sem_g = scr[8:10]
        sem_o = scr[10:12]

        wid = lax.axis_index("s") * _NC + lax.axis_index("c")
        b_base = wid * b_per_w

        def idx_load(sup, si):
            pltpu.async_copy(
                loc_hbm.at[pl.ds((b_base + sup * _SUP) * n_t, _SUP * n_t)],
                idx_v[si], sem_i[si])

        def idx_wait(si):
            pltpu.make_async_copy(
                loc_hbm.at[pl.ds(0, _SUP * n_t)], idx_v[si], sem_i[si]).wait()

        def gathers(r, si, g):
            for off, ln in _SPLITS:
                pltpu.async_copy(
                    table_hbm.at[idx_v[si].at[pl.ds(r * n_t + off, ln)]],
                    gv[g].at[pl.ds(off, ln)],
                    sem_g[g],
                )

        def gathers_wait(g):
            pltpu.make_async_copy(
                table_hbm.at[pl.ds(0, n_t)], gv[g], sem_g[g]).wait()

        def compact(g, rb):
            def body(i8, carry):
                for dr in range(_SUP):
                    r = i8 * _SUP + dr
                    for j in range(dim // _L):
                        rv[rb][0, r, pl.ds(j * _L, _L)] = (
                            gv[g][r, pl.ds(j * _L, _L)])
                return carry
            lax.fori_loop(0, n_t // _SUP, body, 0, unroll=False)

        def out_write(c, rb):
            pltpu.async_copy(
                rv[rb], out_hbm.at[pl.ds(b_base + c, 1)], sem_o[rb])

        def out_wait(rb):
            pltpu.make_async_copy(
                rv[rb], out_hbm.at[pl.ds(0, 1)], sem_o[rb]).wait()

        def step(c, r, si, si_next, last, drain=True):
            g = r % 2
            gathers_wait(g)
            if r == _SUP - 1 and not last:
                idx_wait(si_next)
            if not (last and r == _SUP - 1):
                gathers((r + 1) % _SUP, si if r < _SUP - 1 else si_next,
                        (r + 1) % 2)
            if drain:
                out_wait(g)
            compact(g, g)
            out_write(c, g)

        idx_load(0, 0)
        idx_load(1, 1)
        idx_wait(0)
        gathers(0, 0, 0)
        for r in range(_SUP):
            step(r, r, 0, 1, last=False, drain=(r >= 2))

        assert (n_sup - 2) % 2 == 0

        def pair_block(s2, carry):
            for ds in range(2):
                s = s2 * 2 + 1 + ds
                si = (1 + ds) % 2
                idx_load(s + 1, (si + 1) % 2)
                for r in range(_SUP):
                    c = s * _SUP + r
                    step(c, r, si, (si + 1) % 2, last=False)
            return carry

        lax.fori_loop(0, (n_sup - 2) // 2, pair_block, 0, unroll=False)

        s = n_sup - 1
        si = s % 2
        for r in range(_SUP):
            c = s * _SUP + r
            step(c, r, si, si, last=True)
        out_wait((b_per_w - 2) % 2)
        out_wait((b_per_w - 1) % 2)

    return k


def kernel(location, embedding_matrix):
    n_b, n_t = location.shape
    v, dim = embedding_matrix.shape
    loc_flat = location.astype(jnp.int32).reshape(-1)
    table_pad = jnp.pad(embedding_matrix, ((0, 0), (0, _PAD - dim)))
    return _make_sc_gather(n_b, n_t, dim)(loc_flat, table_pad)
